# baseline (speedup 1.0000x reference)
import functools

import jax
import jax.numpy as jnp
from jax import lax
from jax.experimental import pallas as pl
from jax.experimental.pallas import tpu as pltpu
from jax.experimental.pallas import tpu_sc as plsc

D = 768
E = 16
F = 1536
T = 4096
EP = 16
TS = 512
BLK = 512
NBLK = 32
CAP = NBLK * BLK
NW = 32
TPW = T // NW
RC = 32
NEG = -1e30


def _router_body(x_ref, gw_ref, gb_ref,
                 e0_ref, e1_ref, w0_ref, w1_ref, r0_ref, r1_ref, cnt_ref,
                 carry):
    pid = pl.program_id(0)

    @pl.when(pid == 0)
    def _():
        carry[...] = jnp.zeros_like(carry)

    s = jnp.dot(x_ref[...], gw_ref[...],
                preferred_element_type=jnp.float32) + gb_ref[...]
    lanes = lax.broadcasted_iota(jnp.int32, (TS, EP), 1)
    m0 = jnp.max(s, axis=1, keepdims=True)
    a0 = jnp.min(jnp.where(s == m0, lanes, EP), axis=1, keepdims=True)
    oh0 = lanes == a0
    s2 = jnp.where(oh0, NEG, s)
    m1 = jnp.max(s2, axis=1, keepdims=True)
    a1 = jnp.min(jnp.where(s2 == m1, lanes, EP), axis=1, keepdims=True)
    oh1 = lanes == a1

    d = jnp.exp(m1 - m0)
    w0 = 1.0 / (1.0 + d)
    w1 = d / (1.0 + d)

    rows_i = lax.broadcasted_iota(jnp.int32, (TS, TS), 0)
    cols_i = lax.broadcasted_iota(jnp.int32, (TS, TS), 1)
    tl = (cols_i < rows_i).astype(jnp.float32)
    f0 = oh0.astype(jnp.float32)
    f1 = oh1.astype(jnp.float32)
    cum0 = jnp.dot(tl, f0, preferred_element_type=jnp.float32)
    cum1 = jnp.dot(tl, f1, preferred_element_type=jnp.float32)
    bc0 = jnp.sum(f0, axis=0, keepdims=True)
    bc1 = jnp.sum(f1, axis=0, keepdims=True)
    c = carry[...]
    r0 = jnp.sum((c + cum0) * f0, axis=1, keepdims=True)
    r1 = jnp.sum((c + bc0 + cum1) * f1, axis=1, keepdims=True)
    newc = c + bc0 + bc1
    carry[...] = newc
    cnt_ref[...] = newc

    e0_ref[...] = a0.astype(jnp.int32)
    e1_ref[...] = a1.astype(jnp.int32)
    w0_ref[...] = w0
    w1_ref[...] = w1
    r0_ref[...] = r0.astype(jnp.int32)
    r1_ref[...] = r1.astype(jnp.int32)


def _router(x2, gwp, gbp):
    nb = T // TS
    outs = pl.pallas_call(
        _router_body,
        grid=(nb,),
        in_specs=[
            pl.BlockSpec((TS, D), lambda b: (b, 0)),
            pl.BlockSpec((D, EP), lambda b: (0, 0)),
            pl.BlockSpec((1, EP), lambda b: (0, 0)),
        ],
        out_specs=[
            pl.BlockSpec((TS, 1), lambda b: (b, 0)),
            pl.BlockSpec((TS, 1), lambda b: (b, 0)),
            pl.BlockSpec((TS, 1), lambda b: (b, 0)),
            pl.BlockSpec((TS, 1), lambda b: (b, 0)),
            pl.BlockSpec((TS, 1), lambda b: (b, 0)),
            pl.BlockSpec((TS, 1), lambda b: (b, 0)),
            pl.BlockSpec((1, EP), lambda b: (0, 0)),
        ],
        out_shape=[
            jax.ShapeDtypeStruct((T, 1), jnp.int32),
            jax.ShapeDtypeStruct((T, 1), jnp.int32),
            jax.ShapeDtypeStruct((T, 1), jnp.float32),
            jax.ShapeDtypeStruct((T, 1), jnp.float32),
            jax.ShapeDtypeStruct((T, 1), jnp.int32),
            jax.ShapeDtypeStruct((T, 1), jnp.int32),
            jax.ShapeDtypeStruct((1, EP), jnp.float32),
        ],
        scratch_shapes=[pltpu.VMEM((1, EP), jnp.float32)],
    )(x2, gwp, gbp)
    return outs


def _dispatch(x2, e0, e1, r0, r1, base):
    mesh = plsc.VectorSubcoreMesh(core_axis_name="c", subcore_axis_name="s")

    @functools.partial(
        pl.kernel,
        out_type=[
            jax.ShapeDtypeStruct((CAP, D), jnp.float32),
            jax.ShapeDtypeStruct((T,), jnp.int32),
            jax.ShapeDtypeStruct((T,), jnp.int32),
        ],
        mesh=mesh,
        scratch_types=[
            pltpu.VMEM((TPW,), jnp.int32),
            pltpu.VMEM((TPW,), jnp.int32),
            pltpu.VMEM((TPW,), jnp.int32),
            pltpu.VMEM((TPW,), jnp.int32),
            pltpu.VMEM((16,), jnp.int32),
            pltpu.VMEM((RC,), jnp.int32),
            pltpu.VMEM((RC,), jnp.int32),
            pltpu.VMEM((RC, D), jnp.float32),
            pltpu.SemaphoreType.DMA,
            pltpu.SemaphoreType.DMA,
        ],
    )
    def k(x_hbm, e0_hbm, e1_hbm, r0_hbm, r1_hbm, base_hbm,
          xs_hbm, s0_hbm, s1_hbm,
          e_v, r_v, s0_v, s1_v, tb_v, i0_v, i1_v, rows_v, sem0, sem1):
        wid = lax.axis_index("s") * 2 + lax.axis_index("c")
        t0 = wid * TPW
        pltpu.sync_copy(base_hbm, tb_v)
        base_vec = tb_v[...]
        for kk in range(2):
            eh = e0_hbm if kk == 0 else e1_hbm
            rh = r0_hbm if kk == 0 else r1_hbm
            sh = s0_hbm if kk == 0 else s1_hbm
            sv = s0_v if kk == 0 else s1_v
            pltpu.sync_copy(eh.at[pl.ds(t0, TPW)], e_v)
            pltpu.sync_copy(rh.at[pl.ds(t0, TPW)], r_v)
            for i in range(TPW // 16):
                idx16 = e_v[pl.ds(i * 16, 16)]
                b16 = jnp.take(base_vec, idx16, mode="wrap")
                sv[pl.ds(i * 16, 16)] = b16 + r_v[pl.ds(i * 16, 16)]
            pltpu.sync_copy(sv, sh.at[pl.ds(t0, TPW)])
        for cch in range(TPW // RC):
            for j in range(RC // 16):
                i0_v[pl.ds(j * 16, 16)] = s0_v[pl.ds(cch * RC + j * 16, 16)]
                i1_v[pl.ds(j * 16, 16)] = s1_v[pl.ds(cch * RC + j * 16, 16)]
            pltpu.sync_copy(x_hbm.at[pl.ds(t0 + cch * RC, RC)], rows_v)
            cp0 = pltpu.async_copy(rows_v, xs_hbm.at[i0_v], sem0)
            cp1 = pltpu.async_copy(rows_v, xs_hbm.at[i1_v], sem1)
            cp0.wait()
            cp1.wait()

    return k(x2, e0, e1, r0, r1, base)


def _ffn_body(be_ref, nv_ref, x_ref, w1_ref, b1_ref, w2_ref, b2_ref, y_ref):
    b = pl.program_id(0)

    @pl.when(b < nv_ref[0])
    def _():
        e = be_ref[b]
        h = jnp.dot(x_ref[...], w1_ref[0], preferred_element_type=jnp.float32)
        h = h + b1_ref[0]
        ha = lax.cond(
            e % 2 == 0,
            lambda v: 0.5 * v * (1.0 + lax.erf(v * 0.7071067811865476)),
            lambda v: v * (1.0 / (1.0 + jnp.exp(-v))),
            h)
        y = jnp.dot(ha, w2_ref[0], preferred_element_type=jnp.float32)
        y_ref[...] = y + b2_ref[0]


def _ffn(block_expert, nvalid, xs, W1, b1, W2, b2):
    grid_spec = pltpu.PrefetchScalarGridSpec(
        num_scalar_prefetch=2,
        grid=(NBLK,),
        in_specs=[
            pl.BlockSpec((BLK, D),
                         lambda b, be, nv: (jnp.where(b < nv[0], b, 0), 0)),
            pl.BlockSpec((1, D, F), lambda b, be, nv: (be[b], 0, 0)),
            pl.BlockSpec((1, 1, F), lambda b, be, nv: (be[b], 0, 0)),
            pl.BlockSpec((1, F, D), lambda b, be, nv: (be[b], 0, 0)),
            pl.BlockSpec((1, 1, D), lambda b, be, nv: (be[b], 0, 0)),
        ],
        out_specs=pl.BlockSpec(
            (BLK, D), lambda b, be, nv: (jnp.where(b < nv[0], b, NBLK - 1), 0)),
    )
    return pl.pallas_call(
        _ffn_body,
        grid_spec=grid_spec,
        out_shape=jax.ShapeDtypeStruct((CAP, D), jnp.float32),
    )(block_expert, nvalid, xs, W1, b1.reshape(E, 1, F), W2, b2.reshape(E, 1, D))


CC = 32


def _combine(ys, s0, s1, w0, w1):
    mesh = plsc.VectorSubcoreMesh(core_axis_name="c", subcore_axis_name="s")
    nch = TPW // CC

    @functools.partial(
        pl.kernel,
        out_type=jax.ShapeDtypeStruct((T, D), jnp.float32),
        mesh=mesh,
        scratch_types=[
            pltpu.VMEM((TPW,), jnp.int32),
            pltpu.VMEM((TPW,), jnp.int32),
            pltpu.VMEM((TPW,), jnp.float32),
            pltpu.VMEM((TPW,), jnp.float32),
            pltpu.VMEM((CC,), jnp.int32),
            pltpu.VMEM((CC,), jnp.int32),
            pltpu.VMEM((CC,), jnp.int32),
            pltpu.VMEM((CC,), jnp.int32),
            pltpu.VMEM((CC, D), jnp.float32),
            pltpu.VMEM((CC, D), jnp.float32),
            pltpu.VMEM((CC, D), jnp.float32),
            pltpu.VMEM((CC, D), jnp.float32),
            pltpu.VMEM((CC, D), jnp.float32),
            pltpu.SemaphoreType.DMA,
            pltpu.SemaphoreType.DMA,
            pltpu.SemaphoreType.DMA,
            pltpu.SemaphoreType.DMA,
        ],
    )
    def k(ys_hbm, s0_hbm, s1_hbm, w0_hbm, w1_hbm, out_hbm,
          s0_v, s1_v, w0_v, w1_v,
          i0a, i1a, i0b, i1b, r0a, r1a, r0b, r1b, out_v,
          sem0a, sem1a, sem0b, sem1b):
        wid = lax.axis_index("s") * 2 + lax.axis_index("c")
        t0 = wid * TPW
        pltpu.sync_copy(s0_hbm.at[pl.ds(t0, TPW)], s0_v)
        pltpu.sync_copy(s1_hbm.at[pl.ds(t0, TPW)], s1_v)
        pltpu.sync_copy(w0_hbm.at[pl.ds(t0, TPW)], w0_v)
        pltpu.sync_copy(w1_hbm.at[pl.ds(t0, TPW)], w1_v)

        bufs = [(i0a, i1a, r0a, r1a, sem0a, sem1a),
                (i0b, i1b, r0b, r1b, sem0b, sem1b)]

        def start(cch, i0, i1, rr0, rr1, sm0, sm1):
            i0[...] = s0_v[pl.ds(cch * CC, CC)]
            i1[...] = s1_v[pl.ds(cch * CC, CC)]
            c0 = pltpu.async_copy(ys_hbm.at[i0], rr0, sm0)
            c1 = pltpu.async_copy(ys_hbm.at[i1], rr1, sm1)
            return c0, c1

        cps = start(0, *bufs[0])
        for cch in range(nch):
            i0, i1, rr0, rr1, sm0, sm1 = bufs[cch % 2]
            cps[0].wait()
            cps[1].wait()
            if cch + 1 < nch:
                cps = start(cch + 1, *bufs[(cch + 1) % 2])
            for g in range(CC // 16):
                wv0 = w0_v[pl.ds(cch * CC + g * 16, 16)]
                wv1 = w1_v[pl.ds(cch * CC + g * 16, 16)]
                for li in range(16):
                    l = g * 16 + li
                    lane = jnp.full((16,), li, jnp.int32)
                    a0 = jnp.take(wv0, lane, mode="wrap")
                    a1 = jnp.take(wv1, lane, mode="wrap")

                    def body(j, _):
                        sl = pl.ds(j * 16, 16)
                        out_v[l, sl] = a0 * rr0[l, sl] + a1 * rr1[l, sl]
                        return 0

                    lax.fori_loop(0, D // 16, body, 0, unroll=4)
            pltpu.sync_copy(out_v, out_hbm.at[pl.ds(t0 + cch * CC, CC)])

    return k(ys, s0, s1, w0, w1)


def kernel(x, gate_W, gate_b, W1, b1, W2, b2):
    B, S, Dm = x.shape
    x2 = x.reshape(T, D)
    e0, e1, w0, w1, r0, r1, cnt = _router(x2, gate_W, gate_b.reshape(1, E))
    e0 = e0.reshape(T)
    e1 = e1.reshape(T)
    w0 = w0.reshape(T)
    w1 = w1.reshape(T)
    r0 = r0.reshape(T)
    r1 = r1.reshape(T)

    counts = cnt[0].astype(jnp.int32)
    cap_e = ((counts + BLK - 1) // BLK) * BLK
    base = jnp.concatenate([jnp.zeros((1,), jnp.int32),
                            jnp.cumsum(cap_e)[:-1].astype(jnp.int32)])
    cumblk = jnp.cumsum(cap_e // BLK)
    block_expert = jnp.searchsorted(
        cumblk, jnp.arange(NBLK, dtype=jnp.int32), side="right").astype(jnp.int32)
    block_expert = jnp.minimum(block_expert, E - 1)

    nvalid = cumblk[-1:].astype(jnp.int32)
    xs, s0, s1 = _dispatch(x2, e0, e1, r0, r1, base)
    ys = _ffn(block_expert, nvalid, xs, W1, b1, W2, b2)
    out = _combine(ys, s0, s1, w0, w1)
    return out.reshape(B, S, Dm)


# speedup vs baseline: 1.0045x; 1.0045x over previous
import functools

import jax
import jax.numpy as jnp
from jax import lax
from jax.experimental import pallas as pl
from jax.experimental.pallas import tpu as pltpu
from jax.experimental.pallas import tpu_sc as plsc

D = 768
E = 16
F = 1536
T = 4096
EP = 16
TS = 512
BLK = 512
NBLK = 32
CAP = NBLK * BLK
NW = 32
TPW = T // NW
RC = 32
NEG = -1e30


def _router_body(x_ref, gw_ref, gb_ref,
                 e0_ref, e1_ref, w0_ref, w1_ref, r0_ref, r1_ref,
                 base_ref, be_ref, nv_ref,
                 carry):
    pid = pl.program_id(0)

    @pl.when(pid == 0)
    def _():
        carry[...] = jnp.zeros_like(carry)

    s = jnp.dot(x_ref[...], gw_ref[...],
                preferred_element_type=jnp.float32) + gb_ref[...]
    lanes = lax.broadcasted_iota(jnp.int32, (TS, EP), 1)
    m0 = jnp.max(s, axis=1, keepdims=True)
    a0 = jnp.min(jnp.where(s == m0, lanes, EP), axis=1, keepdims=True)
    oh0 = lanes == a0
    s2 = jnp.where(oh0, NEG, s)
    m1 = jnp.max(s2, axis=1, keepdims=True)
    a1 = jnp.min(jnp.where(s2 == m1, lanes, EP), axis=1, keepdims=True)
    oh1 = lanes == a1

    d = jnp.exp(m1 - m0)
    w0 = 1.0 / (1.0 + d)
    w1 = d / (1.0 + d)

    rows_i = lax.broadcasted_iota(jnp.int32, (TS, TS), 0)
    cols_i = lax.broadcasted_iota(jnp.int32, (TS, TS), 1)
    tl = (cols_i < rows_i).astype(jnp.float32)
    f0 = oh0.astype(jnp.float32)
    f1 = oh1.astype(jnp.float32)
    cum0 = jnp.dot(tl, f0, preferred_element_type=jnp.float32)
    cum1 = jnp.dot(tl, f1, preferred_element_type=jnp.float32)
    bc0 = jnp.sum(f0, axis=0, keepdims=True)
    bc1 = jnp.sum(f1, axis=0, keepdims=True)
    c = carry[...]
    r0 = jnp.sum((c + cum0) * f0, axis=1, keepdims=True)
    r1 = jnp.sum((c + bc0 + cum1) * f1, axis=1, keepdims=True)
    newc = c + bc0 + bc1
    carry[...] = newc

    cnt_i = newc.astype(jnp.int32)
    cap = jnp.bitwise_and(cnt_i + (BLK - 1), ~(BLK - 1))
    capf = cap.astype(jnp.float32)
    er = lax.broadcasted_iota(jnp.int32, (EP, EP), 0)
    ec = lax.broadcasted_iota(jnp.int32, (EP, EP), 1)
    xtri = (er < ec).astype(jnp.float32)
    base_ref[...] = jnp.dot(capf, xtri,
                            preferred_element_type=jnp.float32).astype(jnp.int32)
    nbl = capf * (1.0 / BLK)
    cumblk = jnp.dot(nbl, (er <= ec).astype(jnp.float32),
                     preferred_element_type=jnp.float32)
    bvals = lax.broadcasted_iota(jnp.int32, (NBLK, EP), 0).astype(jnp.float32)
    bex = jnp.sum((jnp.broadcast_to(cumblk, (NBLK, EP)) <= bvals)
                  .astype(jnp.float32), axis=1, keepdims=True)
    be_ref[...] = jnp.minimum(bex.astype(jnp.int32), E - 1)
    nv_ref[...] = jnp.sum(nbl, axis=1, keepdims=True).astype(jnp.int32)

    e0_ref[...] = a0.astype(jnp.int32)
    e1_ref[...] = a1.astype(jnp.int32)
    w0_ref[...] = w0
    w1_ref[...] = w1
    r0_ref[...] = r0.astype(jnp.int32)
    r1_ref[...] = r1.astype(jnp.int32)


def _router(x2, gwp, gbp):
    nb = T // TS
    outs = pl.pallas_call(
        _router_body,
        grid=(nb,),
        in_specs=[
            pl.BlockSpec((TS, D), lambda b: (b, 0)),
            pl.BlockSpec((D, EP), lambda b: (0, 0)),
            pl.BlockSpec((1, EP), lambda b: (0, 0)),
        ],
        out_specs=[
            pl.BlockSpec((TS, 1), lambda b: (b, 0)),
            pl.BlockSpec((TS, 1), lambda b: (b, 0)),
            pl.BlockSpec((TS, 1), lambda b: (b, 0)),
            pl.BlockSpec((TS, 1), lambda b: (b, 0)),
            pl.BlockSpec((TS, 1), lambda b: (b, 0)),
            pl.BlockSpec((TS, 1), lambda b: (b, 0)),
            pl.BlockSpec((1, EP), lambda b: (0, 0)),
            pl.BlockSpec((NBLK, 1), lambda b: (0, 0)),
            pl.BlockSpec((1, 1), lambda b: (0, 0)),
        ],
        out_shape=[
            jax.ShapeDtypeStruct((T, 1), jnp.int32),
            jax.ShapeDtypeStruct((T, 1), jnp.int32),
            jax.ShapeDtypeStruct((T, 1), jnp.float32),
            jax.ShapeDtypeStruct((T, 1), jnp.float32),
            jax.ShapeDtypeStruct((T, 1), jnp.int32),
            jax.ShapeDtypeStruct((T, 1), jnp.int32),
            jax.ShapeDtypeStruct((1, EP), jnp.int32),
            jax.ShapeDtypeStruct((NBLK, 1), jnp.int32),
            jax.ShapeDtypeStruct((1, 1), jnp.int32),
        ],
        scratch_shapes=[pltpu.VMEM((1, EP), jnp.float32)],
    )(x2, gwp, gbp)
    return outs


def _dispatch(x2, e0, e1, r0, r1, base):
    mesh = plsc.VectorSubcoreMesh(core_axis_name="c", subcore_axis_name="s")

    @functools.partial(
        pl.kernel,
        out_type=[
            jax.ShapeDtypeStruct((CAP, D), jnp.float32),
            jax.ShapeDtypeStruct((T,), jnp.int32),
            jax.ShapeDtypeStruct((T,), jnp.int32),
        ],
        mesh=mesh,
        scratch_types=[
            pltpu.VMEM((TPW,), jnp.int32),
            pltpu.VMEM((TPW,), jnp.int32),
            pltpu.VMEM((TPW,), jnp.int32),
            pltpu.VMEM((TPW,), jnp.int32),
            pltpu.VMEM((16,), jnp.int32),
            pltpu.VMEM((RC,), jnp.int32),
            pltpu.VMEM((RC,), jnp.int32),
            pltpu.VMEM((RC, D), jnp.float32),
            pltpu.SemaphoreType.DMA,
            pltpu.SemaphoreType.DMA,
        ],
    )
    def k(x_hbm, e0_hbm, e1_hbm, r0_hbm, r1_hbm, base_hbm,
          xs_hbm, s0_hbm, s1_hbm,
          e_v, r_v, s0_v, s1_v, tb_v, i0_v, i1_v, rows_v, sem0, sem1):
        wid = lax.axis_index("s") * 2 + lax.axis_index("c")
        t0 = wid * TPW
        pltpu.sync_copy(base_hbm.at[0], tb_v)
        base_vec = tb_v[...]
        for kk in range(2):
            eh = e0_hbm if kk == 0 else e1_hbm
            rh = r0_hbm if kk == 0 else r1_hbm
            sh = s0_hbm if kk == 0 else s1_hbm
            sv = s0_v if kk == 0 else s1_v
            pltpu.sync_copy(eh.at[pl.ds(t0, TPW)], e_v)
            pltpu.sync_copy(rh.at[pl.ds(t0, TPW)], r_v)
            for i in range(TPW // 16):
                idx16 = e_v[pl.ds(i * 16, 16)]
                b16 = jnp.take(base_vec, idx16, mode="wrap")
                sv[pl.ds(i * 16, 16)] = b16 + r_v[pl.ds(i * 16, 16)]
            pltpu.sync_copy(sv, sh.at[pl.ds(t0, TPW)])
        for cch in range(TPW // RC):
            for j in range(RC // 16):
                i0_v[pl.ds(j * 16, 16)] = s0_v[pl.ds(cch * RC + j * 16, 16)]
                i1_v[pl.ds(j * 16, 16)] = s1_v[pl.ds(cch * RC + j * 16, 16)]
            pltpu.sync_copy(x_hbm.at[pl.ds(t0 + cch * RC, RC)], rows_v)
            cp0 = pltpu.async_copy(rows_v, xs_hbm.at[i0_v], sem0)
            cp1 = pltpu.async_copy(rows_v, xs_hbm.at[i1_v], sem1)
            cp0.wait()
            cp1.wait()

    return k(x2, e0, e1, r0, r1, base)


def _ffn_body(be_ref, nv_ref, x_ref, w1_ref, b1_ref, w2_ref, b2_ref, y_ref):
    b = pl.program_id(0)

    @pl.when(b < nv_ref[0, 0])
    def _():
        e = be_ref[b, 0]
        h = jnp.dot(x_ref[...], w1_ref[0], preferred_element_type=jnp.float32)
        h = h + b1_ref[0]
        ha = lax.cond(
            e % 2 == 0,
            lambda v: 0.5 * v * (1.0 + lax.erf(v * 0.7071067811865476)),
            lambda v: v * (1.0 / (1.0 + jnp.exp(-v))),
            h)
        y = jnp.dot(ha, w2_ref[0], preferred_element_type=jnp.float32)
        y_ref[...] = y + b2_ref[0]


def _ffn(block_expert, nvalid, xs, W1, b1, W2, b2):
    grid_spec = pltpu.PrefetchScalarGridSpec(
        num_scalar_prefetch=2,
        grid=(NBLK,),
        in_specs=[
            pl.BlockSpec((BLK, D),
                         lambda b, be, nv: (jnp.where(b < nv[0, 0], b, 0), 0)),
            pl.BlockSpec((1, D, F), lambda b, be, nv: (be[b, 0], 0, 0)),
            pl.BlockSpec((1, 1, F), lambda b, be, nv: (be[b, 0], 0, 0)),
            pl.BlockSpec((1, F, D), lambda b, be, nv: (be[b, 0], 0, 0)),
            pl.BlockSpec((1, 1, D), lambda b, be, nv: (be[b, 0], 0, 0)),
        ],
        out_specs=pl.BlockSpec(
            (BLK, D),
            lambda b, be, nv: (jnp.where(b < nv[0, 0], b, NBLK - 1), 0)),
    )
    return pl.pallas_call(
        _ffn_body,
        grid_spec=grid_spec,
        out_shape=jax.ShapeDtypeStruct((CAP, D), jnp.float32),
    )(block_expert, nvalid, xs, W1, b1.reshape(E, 1, F), W2, b2.reshape(E, 1, D))


CC = 16


def _combine(ys, s0, s1, w0, w1):
    mesh = plsc.VectorSubcoreMesh(core_axis_name="c", subcore_axis_name="s")
    nch = TPW // CC

    @functools.partial(
        pl.kernel,
        out_type=jax.ShapeDtypeStruct((T, D), jnp.float32),
        mesh=mesh,
        scratch_types=[
            pltpu.VMEM((TPW,), jnp.int32),
            pltpu.VMEM((TPW,), jnp.int32),
            pltpu.VMEM((TPW,), jnp.float32),
            pltpu.VMEM((TPW,), jnp.float32),
            pltpu.VMEM((CC,), jnp.int32),
            pltpu.VMEM((CC,), jnp.int32),
            pltpu.VMEM((CC,), jnp.int32),
            pltpu.VMEM((CC,), jnp.int32),
            pltpu.VMEM((CC, D), jnp.float32),
            pltpu.VMEM((CC, D), jnp.float32),
            pltpu.VMEM((CC, D), jnp.float32),
            pltpu.VMEM((CC, D), jnp.float32),
            pltpu.VMEM((CC, D), jnp.float32),
            pltpu.SemaphoreType.DMA,
            pltpu.SemaphoreType.DMA,
            pltpu.SemaphoreType.DMA,
            pltpu.SemaphoreType.DMA,
        ],
    )
    def k(ys_hbm, s0_hbm, s1_hbm, w0_hbm, w1_hbm, out_hbm,
          s0_v, s1_v, w0_v, w1_v,
          i0a, i1a, i0b, i1b, r0a, r1a, r0b, r1b, out_v,
          sem0a, sem1a, sem0b, sem1b):
        wid = lax.axis_index("s") * 2 + lax.axis_index("c")
        t0 = wid * TPW
        pltpu.sync_copy(s0_hbm.at[pl.ds(t0, TPW)], s0_v)
        pltpu.sync_copy(s1_hbm.at[pl.ds(t0, TPW)], s1_v)
        pltpu.sync_copy(w0_hbm.at[pl.ds(t0, TPW)], w0_v)
        pltpu.sync_copy(w1_hbm.at[pl.ds(t0, TPW)], w1_v)

        bufs = [(i0a, i1a, r0a, r1a, sem0a, sem1a),
                (i0b, i1b, r0b, r1b, sem0b, sem1b)]

        def start(cch, i0, i1, rr0, rr1, sm0, sm1):
            i0[...] = s0_v[pl.ds(cch * CC, CC)]
            i1[...] = s1_v[pl.ds(cch * CC, CC)]
            c0 = pltpu.async_copy(ys_hbm.at[i0], rr0, sm0)
            c1 = pltpu.async_copy(ys_hbm.at[i1], rr1, sm1)
            return c0, c1

        cps = start(0, *bufs[0])
        for cch in range(nch):
            i0, i1, rr0, rr1, sm0, sm1 = bufs[cch % 2]
            cps[0].wait()
            cps[1].wait()
            if cch + 1 < nch:
                cps = start(cch + 1, *bufs[(cch + 1) % 2])
            wv0 = w0_v[pl.ds(cch * CC, CC)]
            wv1 = w1_v[pl.ds(cch * CC, CC)]
            for l in range(CC):
                lane = jnp.full((16,), l, jnp.int32)
                a0 = jnp.take(wv0, lane, mode="wrap")
                a1 = jnp.take(wv1, lane, mode="wrap")

                def body(j, _):
                    sl = pl.ds(j * 16, 16)
                    out_v[l, sl] = a0 * rr0[l, sl] + a1 * rr1[l, sl]
                    return 0

                lax.fori_loop(0, D // 16, body, 0, unroll=4)
            pltpu.sync_copy(out_v, out_hbm.at[pl.ds(t0 + cch * CC, CC)])

    return k(ys, s0, s1, w0, w1)


def kernel(x, gate_W, gate_b, W1, b1, W2, b2):
    B, S, Dm = x.shape
    x2 = x.reshape(T, D)
    e0, e1, w0, w1, r0, r1, base, block_expert, nvalid = _router(
        x2, gate_W, gate_b.reshape(1, E))
    e0 = e0.reshape(T)
    e1 = e1.reshape(T)
    w0 = w0.reshape(T)
    w1 = w1.reshape(T)
    r0 = r0.reshape(T)
    r1 = r1.reshape(T)

    xs, s0, s1 = _dispatch(x2, e0, e1, r0, r1, base)
    ys = _ffn(block_expert, nvalid, xs, W1, b1, W2, b2)
    out = _combine(ys, s0, s1, w0, w1)
    return out.reshape(B, S, Dm)


# speedup vs baseline: 1.2043x; 1.1989x over previous
import functools

import jax
import jax.numpy as jnp
from jax import lax
from jax.experimental import pallas as pl
from jax.experimental.pallas import tpu as pltpu
from jax.experimental.pallas import tpu_sc as plsc

D = 768
E = 16
F = 1536
T = 4096
EP = 16
TS = 512
BLK = 512
NBLK = 32
CAP = NBLK * BLK
NW = 32
TPW = T // NW
RC = 32
NEG = -1e30


def _router_body(x_ref, gw_ref, gb_ref,
                 e0_ref, e1_ref, w0_ref, w1_ref, r0_ref, r1_ref,
                 base_ref, be_ref, nv_ref,
                 carry):
    pid = pl.program_id(0)

    @pl.when(pid == 0)
    def _():
        carry[...] = jnp.zeros_like(carry)

    s = jnp.dot(x_ref[...], gw_ref[...],
                preferred_element_type=jnp.float32) + gb_ref[...]
    lanes = lax.broadcasted_iota(jnp.int32, (TS, EP), 1)
    m0 = jnp.max(s, axis=1, keepdims=True)
    a0 = jnp.min(jnp.where(s == m0, lanes, EP), axis=1, keepdims=True)
    oh0 = lanes == a0
    s2 = jnp.where(oh0, NEG, s)
    m1 = jnp.max(s2, axis=1, keepdims=True)
    a1 = jnp.min(jnp.where(s2 == m1, lanes, EP), axis=1, keepdims=True)
    oh1 = lanes == a1

    d = jnp.exp(m1 - m0)
    w0 = 1.0 / (1.0 + d)
    w1 = d / (1.0 + d)

    rows_i = lax.broadcasted_iota(jnp.int32, (TS, TS), 0)
    cols_i = lax.broadcasted_iota(jnp.int32, (TS, TS), 1)
    tl = (cols_i < rows_i).astype(jnp.float32)
    f0 = oh0.astype(jnp.float32)
    f1 = oh1.astype(jnp.float32)
    cum0 = jnp.dot(tl, f0, preferred_element_type=jnp.float32)
    cum1 = jnp.dot(tl, f1, preferred_element_type=jnp.float32)
    bc0 = jnp.sum(f0, axis=0, keepdims=True)
    bc1 = jnp.sum(f1, axis=0, keepdims=True)
    c = carry[...]
    r0 = jnp.sum((c + cum0) * f0, axis=1, keepdims=True)
    r1 = jnp.sum((c + bc0 + cum1) * f1, axis=1, keepdims=True)
    newc = c + bc0 + bc1
    carry[...] = newc

    cnt_i = newc.astype(jnp.int32)
    cap = jnp.bitwise_and(cnt_i + (BLK - 1), ~(BLK - 1))
    capf = cap.astype(jnp.float32)
    er = lax.broadcasted_iota(jnp.int32, (EP, EP), 0)
    ec = lax.broadcasted_iota(jnp.int32, (EP, EP), 1)
    xtri = (er < ec).astype(jnp.float32)
    base_ref[...] = jnp.dot(capf, xtri,
                            preferred_element_type=jnp.float32).astype(jnp.int32)
    nbl = capf * (1.0 / BLK)
    cumblk = jnp.dot(nbl, (er <= ec).astype(jnp.float32),
                     preferred_element_type=jnp.float32)
    bvals = lax.broadcasted_iota(jnp.int32, (NBLK, EP), 0).astype(jnp.float32)
    bex = jnp.sum((jnp.broadcast_to(cumblk, (NBLK, EP)) <= bvals)
                  .astype(jnp.float32), axis=1, keepdims=True)
    be_ref[...] = jnp.minimum(bex.astype(jnp.int32), E - 1)
    nv_ref[...] = jnp.sum(nbl, axis=1, keepdims=True).astype(jnp.int32)

    e0_ref[...] = a0.astype(jnp.int32)
    e1_ref[...] = a1.astype(jnp.int32)
    w0_ref[...] = w0
    w1_ref[...] = w1
    r0_ref[...] = r0.astype(jnp.int32)
    r1_ref[...] = r1.astype(jnp.int32)


def _router(x2, gwp, gbp):
    nb = T // TS
    outs = pl.pallas_call(
        _router_body,
        grid=(nb,),
        in_specs=[
            pl.BlockSpec((TS, D), lambda b: (b, 0)),
            pl.BlockSpec((D, EP), lambda b: (0, 0)),
            pl.BlockSpec((1, EP), lambda b: (0, 0)),
        ],
        out_specs=[
            pl.BlockSpec((TS, 1), lambda b: (b, 0)),
            pl.BlockSpec((TS, 1), lambda b: (b, 0)),
            pl.BlockSpec((TS, 1), lambda b: (b, 0)),
            pl.BlockSpec((TS, 1), lambda b: (b, 0)),
            pl.BlockSpec((TS, 1), lambda b: (b, 0)),
            pl.BlockSpec((TS, 1), lambda b: (b, 0)),
            pl.BlockSpec((1, EP), lambda b: (0, 0)),
            pl.BlockSpec((NBLK, 1), lambda b: (0, 0)),
            pl.BlockSpec((1, 1), lambda b: (0, 0)),
        ],
        out_shape=[
            jax.ShapeDtypeStruct((T, 1), jnp.int32),
            jax.ShapeDtypeStruct((T, 1), jnp.int32),
            jax.ShapeDtypeStruct((T, 1), jnp.float32),
            jax.ShapeDtypeStruct((T, 1), jnp.float32),
            jax.ShapeDtypeStruct((T, 1), jnp.int32),
            jax.ShapeDtypeStruct((T, 1), jnp.int32),
            jax.ShapeDtypeStruct((1, EP), jnp.int32),
            jax.ShapeDtypeStruct((NBLK, 1), jnp.int32),
            jax.ShapeDtypeStruct((1, 1), jnp.int32),
        ],
        scratch_shapes=[pltpu.VMEM((1, EP), jnp.float32)],
    )(x2, gwp, gbp)
    return outs


def _dispatch(x2, e0, e1, r0, r1, base):
    mesh = plsc.VectorSubcoreMesh(core_axis_name="c", subcore_axis_name="s")

    @functools.partial(
        pl.kernel,
        out_type=[
            jax.ShapeDtypeStruct((CAP, D), jnp.float32),
            jax.ShapeDtypeStruct((T,), jnp.int32),
            jax.ShapeDtypeStruct((T,), jnp.int32),
        ],
        mesh=mesh,
        scratch_types=[
            pltpu.VMEM((TPW,), jnp.int32),
            pltpu.VMEM((TPW,), jnp.int32),
            pltpu.VMEM((TPW,), jnp.int32),
            pltpu.VMEM((TPW,), jnp.int32),
            pltpu.VMEM((16,), jnp.int32),
            pltpu.VMEM((RC,), jnp.int32),
            pltpu.VMEM((RC,), jnp.int32),
            pltpu.VMEM((RC, D), jnp.float32),
            pltpu.SemaphoreType.DMA,
            pltpu.SemaphoreType.DMA,
        ],
    )
    def k(x_hbm, e0_hbm, e1_hbm, r0_hbm, r1_hbm, base_hbm,
          xs_hbm, s0_hbm, s1_hbm,
          e_v, r_v, s0_v, s1_v, tb_v, i0_v, i1_v, rows_v, sem0, sem1):
        wid = lax.axis_index("s") * 2 + lax.axis_index("c")
        t0 = wid * TPW
        pltpu.sync_copy(base_hbm.at[0], tb_v)
        base_vec = tb_v[...]
        for kk in range(2):
            eh = e0_hbm if kk == 0 else e1_hbm
            rh = r0_hbm if kk == 0 else r1_hbm
            sh = s0_hbm if kk == 0 else s1_hbm
            sv = s0_v if kk == 0 else s1_v
            pltpu.sync_copy(eh.at[pl.ds(t0, TPW)], e_v)
            pltpu.sync_copy(rh.at[pl.ds(t0, TPW)], r_v)
            for i in range(TPW // 16):
                idx16 = e_v[pl.ds(i * 16, 16)]
                b16 = jnp.take(base_vec, idx16, mode="wrap")
                sv[pl.ds(i * 16, 16)] = b16 + r_v[pl.ds(i * 16, 16)]
            pltpu.sync_copy(sv, sh.at[pl.ds(t0, TPW)])
        for cch in range(TPW // RC):
            for j in range(RC // 16):
                i0_v[pl.ds(j * 16, 16)] = s0_v[pl.ds(cch * RC + j * 16, 16)]
                i1_v[pl.ds(j * 16, 16)] = s1_v[pl.ds(cch * RC + j * 16, 16)]
            pltpu.sync_copy(x_hbm.at[pl.ds(t0 + cch * RC, RC)], rows_v)
            cp0 = pltpu.async_copy(rows_v, xs_hbm.at[i0_v], sem0)
            cp1 = pltpu.async_copy(rows_v, xs_hbm.at[i1_v], sem1)
            cp0.wait()
            cp1.wait()

    return k(x2, e0, e1, r0, r1, base)


def _ffn_body(be_ref, nv_ref, x_ref, w1_ref, b1_ref, w2_ref, b2_ref, y_ref):
    b = pl.program_id(0)

    @pl.when(b < nv_ref[0, 0])
    def _():
        e = be_ref[b, 0]
        h = jnp.dot(x_ref[...], w1_ref[0], preferred_element_type=jnp.float32)
        h = h + b1_ref[0]
        g = 0.5 * h * (1.0 + lax.erf(h * 0.7071067811865476))
        sl = h * (1.0 / (1.0 + jnp.exp(-h)))
        ha = jnp.where(e % 2 == 0, g, sl)
        y = jnp.dot(ha, w2_ref[0], preferred_element_type=jnp.float32)
        y_ref[...] = y + b2_ref[0]


def _ffn(block_expert, nvalid, xs, W1, b1, W2, b2):
    grid_spec = pltpu.PrefetchScalarGridSpec(
        num_scalar_prefetch=2,
        grid=(NBLK,),
        in_specs=[
            pl.BlockSpec((BLK, D),
                         lambda b, be, nv: (jnp.where(b < nv[0, 0], b, 0), 0)),
            pl.BlockSpec((1, D, F), lambda b, be, nv: (be[b, 0], 0, 0)),
            pl.BlockSpec((1, 1, F), lambda b, be, nv: (be[b, 0], 0, 0)),
            pl.BlockSpec((1, F, D), lambda b, be, nv: (be[b, 0], 0, 0)),
            pl.BlockSpec((1, 1, D), lambda b, be, nv: (be[b, 0], 0, 0)),
        ],
        out_specs=pl.BlockSpec(
            (BLK, D),
            lambda b, be, nv: (jnp.where(b < nv[0, 0], b, NBLK - 1), 0)),
    )
    return pl.pallas_call(
        _ffn_body,
        grid_spec=grid_spec,
        out_shape=jax.ShapeDtypeStruct((CAP, D), jnp.float32),
    )(block_expert, nvalid, xs, W1, b1.reshape(E, 1, F), W2, b2.reshape(E, 1, D))


CC = 16


def _combine(ys, s0, s1, w0, w1):
    mesh = plsc.VectorSubcoreMesh(core_axis_name="c", subcore_axis_name="s")
    nch = TPW // CC

    @functools.partial(
        pl.kernel,
        out_type=jax.ShapeDtypeStruct((T, D), jnp.float32),
        mesh=mesh,
        scratch_types=[
            pltpu.VMEM((TPW,), jnp.int32),
            pltpu.VMEM((TPW,), jnp.int32),
            pltpu.VMEM((TPW,), jnp.float32),
            pltpu.VMEM((TPW,), jnp.float32),
            pltpu.VMEM((CC,), jnp.int32),
            pltpu.VMEM((CC,), jnp.int32),
            pltpu.VMEM((CC,), jnp.int32),
            pltpu.VMEM((CC,), jnp.int32),
            pltpu.VMEM((CC, D), jnp.float32),
            pltpu.VMEM((CC, D), jnp.float32),
            pltpu.VMEM((CC, D), jnp.float32),
            pltpu.VMEM((CC, D), jnp.float32),
            pltpu.VMEM((CC, D), jnp.float32),
            pltpu.SemaphoreType.DMA,
            pltpu.SemaphoreType.DMA,
            pltpu.SemaphoreType.DMA,
            pltpu.SemaphoreType.DMA,
        ],
    )
    def k(ys_hbm, s0_hbm, s1_hbm, w0_hbm, w1_hbm, out_hbm,
          s0_v, s1_v, w0_v, w1_v,
          i0a, i1a, i0b, i1b, r0a, r1a, r0b, r1b, out_v,
          sem0a, sem1a, sem0b, sem1b):
        wid = lax.axis_index("s") * 2 + lax.axis_index("c")
        t0 = wid * TPW
        pltpu.sync_copy(s0_hbm.at[pl.ds(t0, TPW)], s0_v)
        pltpu.sync_copy(s1_hbm.at[pl.ds(t0, TPW)], s1_v)
        pltpu.sync_copy(w0_hbm.at[pl.ds(t0, TPW)], w0_v)
        pltpu.sync_copy(w1_hbm.at[pl.ds(t0, TPW)], w1_v)

        bufs = [(i0a, i1a, r0a, r1a, sem0a, sem1a),
                (i0b, i1b, r0b, r1b, sem0b, sem1b)]

        def start(cch, i0, i1, rr0, rr1, sm0, sm1):
            i0[...] = s0_v[pl.ds(cch * CC, CC)]
            i1[...] = s1_v[pl.ds(cch * CC, CC)]
            c0 = pltpu.async_copy(ys_hbm.at[i0], rr0, sm0)
            c1 = pltpu.async_copy(ys_hbm.at[i1], rr1, sm1)
            return c0, c1

        cps = start(0, *bufs[0])
        for cch in range(nch):
            i0, i1, rr0, rr1, sm0, sm1 = bufs[cch % 2]
            cps[0].wait()
            cps[1].wait()
            if cch + 1 < nch:
                cps = start(cch + 1, *bufs[(cch + 1) % 2])
            wv0 = w0_v[pl.ds(cch * CC, CC)]
            wv1 = w1_v[pl.ds(cch * CC, CC)]
            for l in range(CC):
                lane = jnp.full((16,), l, jnp.int32)
                a0 = jnp.take(wv0, lane, mode="wrap")
                a1 = jnp.take(wv1, lane, mode="wrap")

                def body(j, _):
                    sl = pl.ds(j * 16, 16)
                    out_v[l, sl] = a0 * rr0[l, sl] + a1 * rr1[l, sl]
                    return 0

                lax.fori_loop(0, D // 16, body, 0, unroll=4)
            pltpu.sync_copy(out_v, out_hbm.at[pl.ds(t0 + cch * CC, CC)])

    return k(ys, s0, s1, w0, w1)


def kernel(x, gate_W, gate_b, W1, b1, W2, b2):
    B, S, Dm = x.shape
    x2 = x.reshape(T, D)
    e0, e1, w0, w1, r0, r1, base, block_expert, nvalid = _router(
        x2, gate_W, gate_b.reshape(1, E))
    e0 = e0.reshape(T)
    e1 = e1.reshape(T)
    w0 = w0.reshape(T)
    w1 = w1.reshape(T)
    r0 = r0.reshape(T)
    r1 = r1.reshape(T)

    xs, s0, s1 = _dispatch(x2, e0, e1, r0, r1, base)
    ys = _ffn(block_expert, nvalid, xs, W1, b1, W2, b2)
    out = _combine(ys, s0, s1, w0, w1)
    return out.reshape(B, S, Dm)
